# baseline (speedup 1.0000x reference)
import functools
import jax
import jax.numpy as jnp
from jax import lax
from jax.experimental import pallas as pl
from jax.experimental.pallas import tpu as pltpu
from jax.experimental.pallas import tpu_sc as plsc

EPS = 1e-12
_NC = 2
_NS = 16
_NW = _NC * _NS
_L = 16
_H = 128
_HV = _H // _L
_C = 128


def _newton_rsqrt(x):
    i = lax.bitcast_convert_type(x, jnp.int32)
    i = jnp.int32(0x5F3759DF) - lax.shift_right_arithmetic(i, 1)
    y = lax.bitcast_convert_type(i, jnp.float32)
    for _ in range(3):
        y = y * (1.5 - 0.5 * x * y * y)
    return y


def _gather_splat(ref, row, lane):
    ridx = jnp.full((_L,), row, dtype=jnp.int32)
    lidx = jnp.full((_L,), lane, dtype=jnp.int32)
    return plsc.load_gather(ref, [ridx, lidx])


def _sc_body(basket_hbm, mask_hbm, pos_hbm, gamma_hbm, beta_hbm, out_hbm,
             pos_v, in0_v, in1_v, out0_v, out1_v, mask_v, gamma_v, beta_v,
             tmp_v, sin0, sin1, sout0, sout1):
    wid = lax.axis_index("s") * _NC + lax.axis_index("c")
    tokens = basket_hbm.shape[0]
    per_w = tokens // _NW
    nchunk = per_w // _C
    t_base = wid * per_w

    pltpu.sync_copy(pos_hbm.at[pl.ds(0, 200)], pos_v)
    pltpu.sync_copy(mask_hbm.at[pl.ds(t_base, per_w)], mask_v)
    pltpu.sync_copy(gamma_hbm, gamma_v)
    pltpu.sync_copy(beta_hbm, beta_v)
    gv = [gamma_v[pl.ds(h * _L, _L)] for h in range(_HV)]
    bv = [beta_v[pl.ds(h * _L, _L)] for h in range(_HV)]
    r0v = [pos_v[0, pl.ds(h * _L, _L)] for h in range(_HV)]

    def posd_body(s, _):
        for h in range(_HV):
            pos_v[s, pl.ds(h * _L, _L)] = pos_v[s, pl.ds(h * _L, _L)] - r0v[h]
        return 0

    lax.fori_loop(1, 200, posd_body, 0)
    for h in range(_HV):
        pos_v[0, pl.ds(h * _L, _L)] = jnp.zeros((_L,), jnp.float32)

    ins = [in0_v, in1_v]
    outs = [out0_v, out1_v]
    sins = [sin0, sin1]
    souts = [sout0, sout1]

    def in_src(k):
        return basket_hbm.at[pl.ds(t_base + k * _C, _C)]

    def out_dst(k):
        return out_hbm.at[pl.ds(t_base + k * _C, _C)]

    def compute_chunk(k, in_v, out_v):
        @plsc.parallel_loop(0, _C, unroll=8)
        def token_body(j):
            tl = k * _C + j
            s = lax.rem(tl, 200)
            jidx = jnp.full((_L,), tl, dtype=jnp.int32)
            m = plsc.load_gather(mask_v, [jidx])
            evs = []
            sumv = jnp.zeros((_L,), jnp.float32)
            sqv = jnp.zeros((_L,), jnp.float32)
            for h in range(_HV):
                b = in_v[j, pl.ds(h * _L, _L)]
                p = pos_v[s, pl.ds(h * _L, _L)]
                e = b + r0v[h] + m * p
                evs.append(e)
                sumv = sumv + e
                sqv = sqv + e * e
            tmp_v[j, pl.ds(0, _L)] = plsc.cumsum(sumv)
            tmp_v[j, pl.ds(_L, _L)] = plsc.cumsum(sqv)
            mean = _gather_splat(tmp_v, j, _L - 1) * (1.0 / _H)
            ex2 = _gather_splat(tmp_v, j, 2 * _L - 1) * (1.0 / _H)
            r = _newton_rsqrt(ex2 - mean * mean + EPS)
            for h in range(_HV):
                out_v[j, pl.ds(h * _L, _L)] = ((evs[h] - mean) * r) * gv[h] + bv[h]

    pltpu.async_copy(in_src(0), ins[0], sins[0])
    pltpu.async_copy(in_src(1), ins[1], sins[1])

    def pipe_body(k2, _):
        for b in range(2):
            k = k2 * 2 + b
            pltpu.make_async_copy(in_src(k), ins[b], sins[b]).wait()

            @pl.when(k2 >= 1)
            def _():
                pltpu.make_async_copy(outs[b], out_dst(k - 2), souts[b]).wait()

            compute_chunk(k, ins[b], outs[b])
            pltpu.async_copy(outs[b], out_dst(k), souts[b])

            @pl.when(k + 2 < nchunk)
            def _():
                pltpu.async_copy(in_src(k + 2), ins[b], sins[b])

        return 0

    lax.fori_loop(0, nchunk // 2, pipe_body, 0)
    for b in range(2):
        pltpu.make_async_copy(outs[b], out_dst(nchunk - 2 + b), souts[b]).wait()


@jax.jit
def _sc_kernel(basket_embeddings, sequence_mask, pos_table, ln_gamma, ln_beta):
    B, S, H = basket_embeddings.shape
    basket2 = basket_embeddings.reshape(B * S, H)
    mask1 = sequence_mask.reshape(B * S).astype(jnp.float32)
    mesh = plsc.VectorSubcoreMesh(core_axis_name="c", subcore_axis_name="s")
    k = functools.partial(
        pl.kernel,
        mesh=mesh,
        out_type=jax.ShapeDtypeStruct((B * S, H), jnp.float32),
        scratch_types=[
            pltpu.VMEM((200, H), jnp.float32),
            pltpu.VMEM((_C, H), jnp.float32),
            pltpu.VMEM((_C, H), jnp.float32),
            pltpu.VMEM((_C, H), jnp.float32),
            pltpu.VMEM((_C, H), jnp.float32),
            pltpu.VMEM((B * S // _NW,), jnp.float32),
            pltpu.VMEM((H,), jnp.float32),
            pltpu.VMEM((H,), jnp.float32),
            pltpu.VMEM((_C, 2 * _L), jnp.float32),
            pltpu.SemaphoreType.DMA,
            pltpu.SemaphoreType.DMA,
            pltpu.SemaphoreType.DMA,
            pltpu.SemaphoreType.DMA,
        ],
        compiler_params=pltpu.CompilerParams(needs_layout_passes=False),
    )(_sc_body)
    out = k(basket2, mask1, pos_table, ln_gamma, ln_beta)
    return out.reshape(B, S, H)


def kernel(basket_embeddings, sequence_mask, pos_table, ln_gamma, ln_beta):
    return _sc_kernel(basket_embeddings, sequence_mask, pos_table,
                      ln_gamma, ln_beta)


# speedup vs baseline: 1.0047x; 1.0047x over previous
import functools
import jax
import jax.numpy as jnp
from jax import lax
from jax.experimental import pallas as pl
from jax.experimental.pallas import tpu as pltpu
from jax.experimental.pallas import tpu_sc as plsc

EPS = 1e-12
_NC = 2
_NS = 16
_NW = _NC * _NS
_L = 16
_H = 128
_HV = _H // _L
_C = 64


def _newton_rsqrt(x):
    i = lax.bitcast_convert_type(x, jnp.int32)
    i = jnp.int32(0x5F3759DF) - lax.shift_right_arithmetic(i, 1)
    y = lax.bitcast_convert_type(i, jnp.float32)
    for _ in range(3):
        y = y * (1.5 - 0.5 * x * y * y)
    return y


def _gather_splat(ref, row, lane):
    ridx = jnp.full((_L,), row, dtype=jnp.int32)
    lidx = jnp.full((_L,), lane, dtype=jnp.int32)
    return plsc.load_gather(ref, [ridx, lidx])


def _sc_body(basket_hbm, mask_hbm, pos_hbm, gamma_hbm, beta_hbm, out_hbm,
             pos_v, in0_v, in1_v, out0_v, out1_v, mask_v, gamma_v, beta_v,
             tmp_v, sin0, sin1, sout0, sout1):
    wid = lax.axis_index("s") * _NC + lax.axis_index("c")
    tokens = basket_hbm.shape[0]
    per_w = tokens // _NW
    nchunk = per_w // _C
    t_base = wid * per_w

    pltpu.sync_copy(pos_hbm.at[pl.ds(0, 200)], pos_v)
    pltpu.sync_copy(mask_hbm.at[pl.ds(t_base, per_w)], mask_v)
    pltpu.sync_copy(gamma_hbm, gamma_v)
    pltpu.sync_copy(beta_hbm, beta_v)
    gv = [gamma_v[pl.ds(h * _L, _L)] for h in range(_HV)]
    bv = [beta_v[pl.ds(h * _L, _L)] for h in range(_HV)]
    r0v = [pos_v[0, pl.ds(h * _L, _L)] for h in range(_HV)]

    def posd_body(s, _):
        for h in range(_HV):
            pos_v[s, pl.ds(h * _L, _L)] = pos_v[s, pl.ds(h * _L, _L)] - r0v[h]
        return 0

    lax.fori_loop(1, 200, posd_body, 0)
    for h in range(_HV):
        pos_v[0, pl.ds(h * _L, _L)] = jnp.zeros((_L,), jnp.float32)

    ins = [in0_v, in1_v]
    outs = [out0_v, out1_v]
    sins = [sin0, sin1]
    souts = [sout0, sout1]

    def in_src(k):
        return basket_hbm.at[pl.ds(t_base + k * _C, _C)]

    def out_dst(k):
        return out_hbm.at[pl.ds(t_base + k * _C, _C)]

    def compute_chunk(k, in_v, out_v):
        @plsc.parallel_loop(0, _C, unroll=8)
        def token_body(j):
            tl = k * _C + j
            s = lax.rem(tl, 200)
            jidx = jnp.full((_L,), tl, dtype=jnp.int32)
            m = plsc.load_gather(mask_v, [jidx])
            evs = []
            sumv = jnp.zeros((_L,), jnp.float32)
            sqv = jnp.zeros((_L,), jnp.float32)
            for h in range(_HV):
                b = in_v[j, pl.ds(h * _L, _L)]
                p = pos_v[s, pl.ds(h * _L, _L)]
                e = b + r0v[h] + m * p
                evs.append(e)
                sumv = sumv + e
                sqv = sqv + e * e
            tmp_v[j, pl.ds(0, _L)] = plsc.cumsum(sumv)
            tmp_v[j, pl.ds(_L, _L)] = plsc.cumsum(sqv)
            mean = _gather_splat(tmp_v, j, _L - 1) * (1.0 / _H)
            ex2 = _gather_splat(tmp_v, j, 2 * _L - 1) * (1.0 / _H)
            r = _newton_rsqrt(ex2 - mean * mean + EPS)
            for h in range(_HV):
                out_v[j, pl.ds(h * _L, _L)] = ((evs[h] - mean) * r) * gv[h] + bv[h]

    pltpu.async_copy(in_src(0), ins[0], sins[0])
    pltpu.async_copy(in_src(1), ins[1], sins[1])

    def pipe_body(k2, _):
        for b in range(2):
            k = k2 * 2 + b
            pltpu.make_async_copy(in_src(k), ins[b], sins[b]).wait()

            @pl.when(k2 >= 1)
            def _():
                pltpu.make_async_copy(outs[b], out_dst(k - 2), souts[b]).wait()

            compute_chunk(k, ins[b], outs[b])
            pltpu.async_copy(outs[b], out_dst(k), souts[b])

            @pl.when(k + 2 < nchunk)
            def _():
                pltpu.async_copy(in_src(k + 2), ins[b], sins[b])

        return 0

    lax.fori_loop(0, nchunk // 2, pipe_body, 0)
    for b in range(2):
        pltpu.make_async_copy(outs[b], out_dst(nchunk - 2 + b), souts[b]).wait()


@jax.jit
def _sc_kernel(basket_embeddings, sequence_mask, pos_table, ln_gamma, ln_beta):
    B, S, H = basket_embeddings.shape
    basket2 = basket_embeddings.reshape(B * S, H)
    mask1 = sequence_mask.reshape(B * S).astype(jnp.float32)
    mesh = plsc.VectorSubcoreMesh(core_axis_name="c", subcore_axis_name="s")
    k = functools.partial(
        pl.kernel,
        mesh=mesh,
        out_type=jax.ShapeDtypeStruct((B * S, H), jnp.float32),
        scratch_types=[
            pltpu.VMEM((200, H), jnp.float32),
            pltpu.VMEM((_C, H), jnp.float32),
            pltpu.VMEM((_C, H), jnp.float32),
            pltpu.VMEM((_C, H), jnp.float32),
            pltpu.VMEM((_C, H), jnp.float32),
            pltpu.VMEM((B * S // _NW,), jnp.float32),
            pltpu.VMEM((H,), jnp.float32),
            pltpu.VMEM((H,), jnp.float32),
            pltpu.VMEM((_C, 2 * _L), jnp.float32),
            pltpu.SemaphoreType.DMA,
            pltpu.SemaphoreType.DMA,
            pltpu.SemaphoreType.DMA,
            pltpu.SemaphoreType.DMA,
        ],
        compiler_params=pltpu.CompilerParams(needs_layout_passes=False),
    )(_sc_body)
    out = k(basket2, mask1, pos_table, ln_gamma, ln_beta)
    return out.reshape(B, S, H)


def kernel(basket_embeddings, sequence_mask, pos_table, ln_gamma, ln_beta):
    return _sc_kernel(basket_embeddings, sequence_mask, pos_table,
                      ln_gamma, ln_beta)
